# baseline (speedup 1.0000x reference)
import jax
import jax.numpy as jnp
from jax import lax
from jax.experimental import pallas as pl
from jax.experimental.pallas import tpu as pltpu

B = 4096
D = 64
PLEN = 20
PPAD = 32
NQ = 8


def _body(cw_ref, p0i_ref, w0_ref, w1_ref, labels_ref, out_ref,
          rows, p0b, sem, psem):
    for j in range(PLEN):
        pltpu.make_async_copy(w1_ref.at[pl.ds(p0i_ref[j], 1)],
                              p0b.at[pl.ds(j, 1)], psem).start()

    def issue(step, _):
        for j in range(NQ):
            b = step * NQ + j
            pltpu.make_async_copy(w0_ref.at[pl.ds(cw_ref[b], 1)],
                                  rows.at[pl.ds(b, 1)], sem.at[j]).start()
        return 0

    lax.fori_loop(0, B // NQ, issue, 0)
    for j in range(NQ):
        pltpu.make_async_copy(w0_ref.at[pl.ds(0, B // NQ)],
                              rows.at[pl.ds(0, B // NQ)], sem.at[j]).wait()
    pltpu.make_async_copy(w1_ref.at[pl.ds(0, PLEN)],
                          p0b.at[pl.ds(0, PLEN)], psem).wait()

    c = rows[...]
    p = p0b[...]
    scores = lax.dot_general(c, p, (((1,), (1,)), ((), ())),
                             preferred_element_type=jnp.float32)
    s = scores[:, :PLEN]
    lab = labels_ref[...]
    z = jnp.log(1.0 / (1.0 + jnp.exp(-s)))
    log_z = jnp.maximum(jnp.log(z), -100.0)
    log_1mz = jnp.maximum(jnp.log(1.0 - z), -100.0)
    out_ref[0, 0] = -jnp.sum(lab * log_z + (1.0 - lab) * log_1mz)


def kernel(c_words, paths, labels, W0, W1):
    c_words = jnp.squeeze(c_words).astype(jnp.int32)
    paths0 = jnp.squeeze(paths)[0].astype(jnp.int32)
    labels = jnp.squeeze(labels)
    out = pl.pallas_call(
        _body,
        out_shape=jax.ShapeDtypeStruct((1, 1), jnp.float32),
        in_specs=[
            pl.BlockSpec(memory_space=pltpu.SMEM),
            pl.BlockSpec(memory_space=pltpu.SMEM),
            pl.BlockSpec(memory_space=pl.ANY),
            pl.BlockSpec(memory_space=pl.ANY),
            pl.BlockSpec(memory_space=pltpu.VMEM),
        ],
        out_specs=pl.BlockSpec(memory_space=pltpu.SMEM),
        scratch_shapes=[
            pltpu.VMEM((B, D), jnp.float32),
            pltpu.VMEM((PPAD, D), jnp.float32),
            pltpu.SemaphoreType.DMA((NQ,)),
            pltpu.SemaphoreType.DMA,
        ],
    )(c_words, paths0, W0, W1, labels)
    return out[0, 0]
